# Initial kernel scaffold; baseline (speedup 1.0000x reference)
import jax
import jax.numpy as jnp
from jax.experimental import pallas as pl


def kernel(x, virtual_node, batch, W1, b1, g1, be1, W2, b2, g2, be2, res_w):
    raise NotImplementedError("write your pallas kernel here")


# speedup vs baseline: 6.3878x; 6.3878x over previous
import functools

import jax
import jax.numpy as jnp
from jax.experimental import pallas as pl
from jax.experimental.pallas import tpu as pltpu

N = 50000
D = 512
B = 64
BLOCK = 2000
GRID = N // BLOCK


def _body(x_ref, vn_ref, batch_ref, W1_ref, b1_ref, g1_ref, be1_ref,
          W2_ref, b2_ref, g2_ref, be2_ref, rw_ref,
          xout_ref, vnout_ref, sums_ref, counts_ref):
    i = pl.program_id(0)
    b = batch_ref[0, 0, :]
    seg_iota = jax.lax.broadcasted_iota(jnp.int32, (BLOCK, B), 1)
    onehot = (b[:, None] == seg_iota).astype(jnp.float32)

    xb = x_ref[...]
    gathered = jax.lax.dot_general(
        onehot, vn_ref[...], (((1,), (0,)), ((), ())),
        preferred_element_type=jnp.float32,
        precision=jax.lax.Precision.HIGHEST)
    xo = xb + gathered
    xout_ref[...] = xo

    psums = jax.lax.dot_general(
        onehot, xo, (((0,), (0,)), ((), ())),
        preferred_element_type=jnp.float32,
        precision=jax.lax.Precision.HIGHEST)
    pcounts = jnp.sum(onehot, axis=0)

    @pl.when(i == 0)
    def _init():
        sums_ref[...] = psums
        counts_ref[0, :] = pcounts

    @pl.when(i > 0)
    def _acc():
        sums_ref[...] += psums
        counts_ref[0, :] += pcounts

    @pl.when(i == GRID - 1)
    def _mlp():
        counts = counts_ref[0, :]
        mean = sums_ref[...] * (1.0 / jnp.maximum(counts, 1.0))[:, None]

        def dense(h, W_ref, bias_ref):
            return jax.lax.dot_general(
                h, W_ref[...], (((1,), (1,)), ((), ())),
                preferred_element_type=jnp.float32,
                precision=jax.lax.Precision.HIGHEST) + bias_ref[0, :]

        def bn(h, g_ref, be_ref):
            mu = jnp.mean(h, axis=0)
            var = jnp.mean((h - mu) ** 2, axis=0)
            return (h - mu) / jnp.sqrt(var + 1e-5) * g_ref[0, :] + be_ref[0, :]

        h = dense(mean, W1_ref, b1_ref)
        h = jnp.maximum(bn(h, g1_ref, be1_ref), 0.0)
        h = dense(h, W2_ref, b2_ref)
        vn_upd = jnp.maximum(bn(h, g2_ref, be2_ref), 0.0)
        alpha = jax.nn.sigmoid(rw_ref[0, 0])
        vnout_ref[...] = alpha * vn_ref[...] + (1.0 - alpha) * vn_upd


@jax.jit
def kernel(x, virtual_node, batch, W1, b1, g1, be1, W2, b2, g2, be2, res_w):
    batch3 = batch.astype(jnp.int32).reshape(GRID, 1, BLOCK)
    row = lambda v: v.reshape(1, D)
    full = lambda shape: pl.BlockSpec(shape, lambda i: (0,) * len(shape))
    x_out, vn_out = pl.pallas_call(
        _body,
        grid=(GRID,),
        in_specs=[
            pl.BlockSpec((BLOCK, D), lambda i: (i, 0)),
            full((B, D)),
            pl.BlockSpec((1, 1, BLOCK), lambda i: (i, 0, 0)),
            full((D, D)), full((1, D)), full((1, D)), full((1, D)),
            full((D, D)), full((1, D)), full((1, D)), full((1, D)),
            pl.BlockSpec(memory_space=pltpu.SMEM),
        ],
        out_specs=[
            pl.BlockSpec((BLOCK, D), lambda i: (i, 0)),
            full((B, D)),
        ],
        out_shape=[
            jax.ShapeDtypeStruct((N, D), jnp.float32),
            jax.ShapeDtypeStruct((B, D), jnp.float32),
        ],
        scratch_shapes=[
            pltpu.VMEM((B, D), jnp.float32),
            pltpu.VMEM((1, B), jnp.float32),
        ],
        compiler_params=pltpu.CompilerParams(
            dimension_semantics=("arbitrary",),
        ),
    )(x, virtual_node, batch3, W1, row(b1), row(g1), row(be1),
      W2, row(b2), row(g2), row(be2), res_w.reshape(1, 1))
    return (x_out, vn_out)


# speedup vs baseline: 13.4531x; 2.1060x over previous
import functools

import jax
import jax.numpy as jnp
from jax.experimental import pallas as pl
from jax.experimental.pallas import tpu as pltpu

N = 50000
D = 512
B = 64
BLOCK = 2000
GRID = N // BLOCK


def _body(x_ref, vn_ref, batch_ref, W1_ref, b1_ref, g1_ref, be1_ref,
          W2_ref, b2_ref, g2_ref, be2_ref, rw_ref,
          xout_ref, vnout_ref, sums_ref, counts_ref):
    i = pl.program_id(0)
    b = batch_ref[0, 0, :]
    seg_iota = jax.lax.broadcasted_iota(jnp.int32, (BLOCK, B), 1)
    onehot = (b[:, None] == seg_iota).astype(jnp.float32)

    xb = x_ref[...]
    gathered = jax.lax.dot_general(
        onehot, vn_ref[...], (((1,), (0,)), ((), ())),
        preferred_element_type=jnp.float32)
    xo = xb + gathered
    xout_ref[...] = xo

    psums = jax.lax.dot_general(
        onehot, xo, (((0,), (0,)), ((), ())),
        preferred_element_type=jnp.float32)
    pcounts = jnp.sum(onehot, axis=0)

    @pl.when(i == 0)
    def _init():
        sums_ref[...] = psums
        counts_ref[0, :] = pcounts

    @pl.when(i > 0)
    def _acc():
        sums_ref[...] += psums
        counts_ref[0, :] += pcounts

    @pl.when(i == GRID - 1)
    def _mlp():
        counts = counts_ref[0, :]
        mean = sums_ref[...] * (1.0 / jnp.maximum(counts, 1.0))[:, None]

        def dense(h, W_ref, bias_ref):
            return jax.lax.dot_general(
                h, W_ref[...], (((1,), (1,)), ((), ())),
                preferred_element_type=jnp.float32,
                precision=jax.lax.Precision.HIGHEST) + bias_ref[0, :]

        def bn(h, g_ref, be_ref):
            mu = jnp.mean(h, axis=0)
            var = jnp.mean((h - mu) ** 2, axis=0)
            return (h - mu) / jnp.sqrt(var + 1e-5) * g_ref[0, :] + be_ref[0, :]

        h = dense(mean, W1_ref, b1_ref)
        h = jnp.maximum(bn(h, g1_ref, be1_ref), 0.0)
        h = dense(h, W2_ref, b2_ref)
        vn_upd = jnp.maximum(bn(h, g2_ref, be2_ref), 0.0)
        alpha = jax.nn.sigmoid(rw_ref[0, 0])
        vnout_ref[...] = alpha * vn_ref[...] + (1.0 - alpha) * vn_upd


@jax.jit
def kernel(x, virtual_node, batch, W1, b1, g1, be1, W2, b2, g2, be2, res_w):
    batch3 = batch.astype(jnp.int32).reshape(GRID, 1, BLOCK)
    row = lambda v: v.reshape(1, D)
    full = lambda shape: pl.BlockSpec(shape, lambda i: (0,) * len(shape))
    x_out, vn_out = pl.pallas_call(
        _body,
        grid=(GRID,),
        in_specs=[
            pl.BlockSpec((BLOCK, D), lambda i: (i, 0)),
            full((B, D)),
            pl.BlockSpec((1, 1, BLOCK), lambda i: (i, 0, 0)),
            full((D, D)), full((1, D)), full((1, D)), full((1, D)),
            full((D, D)), full((1, D)), full((1, D)), full((1, D)),
            pl.BlockSpec(memory_space=pltpu.SMEM),
        ],
        out_specs=[
            pl.BlockSpec((BLOCK, D), lambda i: (i, 0)),
            full((B, D)),
        ],
        out_shape=[
            jax.ShapeDtypeStruct((N, D), jnp.float32),
            jax.ShapeDtypeStruct((B, D), jnp.float32),
        ],
        scratch_shapes=[
            pltpu.VMEM((B, D), jnp.float32),
            pltpu.VMEM((1, B), jnp.float32),
        ],
        compiler_params=pltpu.CompilerParams(
            dimension_semantics=("arbitrary",),
        ),
    )(x, virtual_node, batch3, W1, row(b1), row(g1), row(be1),
      W2, row(b2), row(g2), row(be2), res_w.reshape(1, 1))
    return (x_out, vn_out)


# speedup vs baseline: 13.6061x; 1.0114x over previous
import functools

import jax
import jax.numpy as jnp
from jax.experimental import pallas as pl
from jax.experimental.pallas import tpu as pltpu

N = 50000
D = 512
B = 64
BLOCK = 5000
GRID = N // BLOCK


def _body(x_ref, vn_ref, batch_ref, W1_ref, b1_ref, g1_ref, be1_ref,
          W2_ref, b2_ref, g2_ref, be2_ref, rw_ref,
          xout_ref, vnout_ref, sums_ref, counts_ref):
    i = pl.program_id(0)
    b = batch_ref[0, 0, :]
    seg_iota = jax.lax.broadcasted_iota(jnp.int32, (BLOCK, B), 1)
    onehot = (b[:, None] == seg_iota).astype(jnp.float32)

    xb = x_ref[...]
    gathered = jax.lax.dot_general(
        onehot, vn_ref[...], (((1,), (0,)), ((), ())),
        preferred_element_type=jnp.float32)
    xo = xb + gathered
    xout_ref[...] = xo

    psums = jax.lax.dot_general(
        onehot, xo, (((0,), (0,)), ((), ())),
        preferred_element_type=jnp.float32)
    pcounts = jnp.sum(onehot, axis=0)

    @pl.when(i == 0)
    def _init():
        sums_ref[...] = psums
        counts_ref[0, :] = pcounts

    @pl.when(i > 0)
    def _acc():
        sums_ref[...] += psums
        counts_ref[0, :] += pcounts

    @pl.when(i == GRID - 1)
    def _mlp():
        counts = counts_ref[0, :]
        mean = sums_ref[...] * (1.0 / jnp.maximum(counts, 1.0))[:, None]

        def dense(h, W_ref, bias_ref):
            return jax.lax.dot_general(
                h, W_ref[...], (((1,), (1,)), ((), ())),
                preferred_element_type=jnp.float32,
                precision=jax.lax.Precision.HIGHEST) + bias_ref[0, :]

        def bn(h, g_ref, be_ref):
            mu = jnp.mean(h, axis=0)
            var = jnp.mean((h - mu) ** 2, axis=0)
            return (h - mu) / jnp.sqrt(var + 1e-5) * g_ref[0, :] + be_ref[0, :]

        h = dense(mean, W1_ref, b1_ref)
        h = jnp.maximum(bn(h, g1_ref, be1_ref), 0.0)
        h = dense(h, W2_ref, b2_ref)
        vn_upd = jnp.maximum(bn(h, g2_ref, be2_ref), 0.0)
        alpha = jax.nn.sigmoid(rw_ref[0, 0])
        vnout_ref[...] = alpha * vn_ref[...] + (1.0 - alpha) * vn_upd


@jax.jit
def kernel(x, virtual_node, batch, W1, b1, g1, be1, W2, b2, g2, be2, res_w):
    batch3 = batch.astype(jnp.int32).reshape(GRID, 1, BLOCK)
    row = lambda v: v.reshape(1, D)
    full = lambda shape: pl.BlockSpec(shape, lambda i: (0,) * len(shape))
    x_out, vn_out = pl.pallas_call(
        _body,
        grid=(GRID,),
        in_specs=[
            pl.BlockSpec((BLOCK, D), lambda i: (i, 0)),
            full((B, D)),
            pl.BlockSpec((1, 1, BLOCK), lambda i: (i, 0, 0)),
            full((D, D)), full((1, D)), full((1, D)), full((1, D)),
            full((D, D)), full((1, D)), full((1, D)), full((1, D)),
            pl.BlockSpec(memory_space=pltpu.SMEM),
        ],
        out_specs=[
            pl.BlockSpec((BLOCK, D), lambda i: (i, 0)),
            full((B, D)),
        ],
        out_shape=[
            jax.ShapeDtypeStruct((N, D), jnp.float32),
            jax.ShapeDtypeStruct((B, D), jnp.float32),
        ],
        scratch_shapes=[
            pltpu.VMEM((B, D), jnp.float32),
            pltpu.VMEM((1, B), jnp.float32),
        ],
        compiler_params=pltpu.CompilerParams(
            dimension_semantics=("arbitrary",),
        ),
    )(x, virtual_node, batch3, W1, row(b1), row(g1), row(be1),
      W2, row(b2), row(g2), row(be2), res_w.reshape(1, 1))
    return (x_out, vn_out)
